# baseline (speedup 1.0000x reference)
import jax
import jax.numpy as jnp
from jax.experimental import pallas as pl
from jax.experimental.pallas import tpu as pltpu

_CR = 80
_NF = 12
_EP = 1008


def _mm(a, b):
    dn = (((1,), (0,)), ((), ()))
    return jax.lax.dot_general(a, b, dn, preferred_element_type=jnp.float32)


def _xw_mm(x, w):
    xh = x.astype(jnp.bfloat16)
    xl = (x - xh.astype(jnp.float32)).astype(jnp.bfloat16)
    wh = w.astype(jnp.bfloat16)
    wl = (w - wh.astype(jnp.float32)).astype(jnp.bfloat16)
    return _mm(xh, wh) + _mm(xh, wl) + _mm(xl, wh)


def _tb(v):
    return jnp.swapaxes(v.astype(jnp.bfloat16), 0, 1)


_SPLIT = 560


def _body(x0b_ref, bt_ref, w0_ref, w1_ref, out0_ref, out1_ref, btc_ref, x1_ref):
    i = pl.program_id(0)
    n_nodes = bt_ref.shape[1]
    ch = x0b_ref.shape[1]
    n_edges = out1_ref.shape[0]

    @pl.when(i < _NF)
    def _build():
        btc_ref[pl.ds(i * _CR, _CR), :] = bt_ref[...].astype(jnp.bfloat16)

    @pl.when(i == 7)
    def _dot1a():
        x1_ref[pl.ds(0, _SPLIT), :] = _mm(
            btc_ref[pl.ds(0, _SPLIT), :], x0b_ref[...])

    @pl.when(i == _NF)
    def _tail():
        tail = n_edges - _NF * _CR
        blk = bt_ref[pl.ds(0, tail), :].astype(jnp.bfloat16)
        btc_ref[pl.ds(_NF * _CR, _EP - _NF * _CR), :] = jnp.concatenate(
            [blk, jnp.zeros((_EP - n_edges, n_nodes), jnp.bfloat16)], axis=0)

    @pl.when(i == _NF + 1)
    def _compute():
        bt = btc_ref[...]
        x1_ref[pl.ds(_SPLIT, _EP - _SPLIT), :] = _mm(
            btc_ref[pl.ds(_SPLIT, _EP - _SPLIT), :], x0b_ref[...])
        y = _xw_mm(x1_ref[...], w0_ref[...])
        x0pT = _mm(_tb(y), bt)
        x0pb = jnp.swapaxes(x0pT.astype(jnp.bfloat16), 0, 1)
        x1p = _mm(bt, x0pb)
        out1_ref[...] = jax.lax.slice(x1p, (0, 0), (n_edges, ch))
        y2 = _xw_mm(x1p, w1_ref[...])
        out0T = _mm(_tb(y2), bt)
        out0_ref[...] = jnp.swapaxes(out0T, 0, 1)


def kernel(x_0, incidence_1, W0, W1):
    n_nodes, ch = x_0.shape
    n_edges = incidence_1.shape[1]
    bt = jnp.swapaxes(incidence_1, 0, 1)
    x0b = x_0.astype(jnp.bfloat16)
    return pl.pallas_call(
        _body,
        grid=(_NF + 2,),
        in_specs=[
            pl.BlockSpec((n_nodes, ch), lambda i: (0, 0)),
            pl.BlockSpec((_CR, n_nodes), lambda i: (jnp.minimum(i, _NF), 0)),
            pl.BlockSpec((ch, ch), lambda i: (0, 0)),
            pl.BlockSpec((ch, ch), lambda i: (0, 0)),
        ],
        out_specs=(
            pl.BlockSpec((n_nodes, ch), lambda i: (0, 0)),
            pl.BlockSpec((n_edges, ch), lambda i: (0, 0)),
        ),
        out_shape=(
            jax.ShapeDtypeStruct((n_nodes, ch), jnp.float32),
            jax.ShapeDtypeStruct((n_edges, ch), jnp.float32),
        ),
        scratch_shapes=[
            pltpu.VMEM((_EP, n_nodes), jnp.bfloat16),
            pltpu.VMEM((_EP, ch), jnp.float32),
        ],
    )(x0b, bt, W0, W1)


# speedup vs baseline: 1.0536x; 1.0536x over previous
import jax
import jax.numpy as jnp
from jax.experimental import pallas as pl
from jax.experimental.pallas import tpu as pltpu

_CR = 80
_NF = 12
_EP = 1008


def _mm(a, b):
    dn = (((1,), (0,)), ((), ()))
    return jax.lax.dot_general(a, b, dn, preferred_element_type=jnp.float32)


def _xw_mm(x, w):
    xh = x.astype(jnp.bfloat16)
    xl = (x - xh.astype(jnp.float32)).astype(jnp.bfloat16)
    wh = w.astype(jnp.bfloat16)
    wl = (w - wh.astype(jnp.float32)).astype(jnp.bfloat16)
    return _mm(xh, wh) + _mm(xh, wl) + _mm(xl, wh)


def _tb(v):
    return jnp.swapaxes(v.astype(jnp.bfloat16), 0, 1)


def _body(x0b_ref, bt_ref, w0_ref, w1_ref, out0_ref, out1_ref, btc_ref):
    i = pl.program_id(0)
    n_nodes = bt_ref.shape[1]
    ch = x0b_ref.shape[1]
    n_edges = out1_ref.shape[0]

    @pl.when(i < _NF)
    def _build():
        btc_ref[pl.ds(i * _CR, _CR), :] = bt_ref[...].astype(jnp.bfloat16)

    @pl.when(i == _NF)
    def _tail():
        tail = n_edges - _NF * _CR
        blk = bt_ref[pl.ds(0, tail), :].astype(jnp.bfloat16)
        btc_ref[pl.ds(_NF * _CR, _EP - _NF * _CR), :] = jnp.concatenate(
            [blk, jnp.zeros((_EP - n_edges, n_nodes), jnp.bfloat16)], axis=0)

    @pl.when(i == _NF + 1)
    def _compute():
        bt = btc_ref[...]
        x1 = _mm(bt, x0b_ref[...])
        y = _xw_mm(x1, w0_ref[...])
        x0pT = _mm(_tb(y), bt)
        x0pb = jnp.swapaxes(x0pT.astype(jnp.bfloat16), 0, 1)
        x1p = _mm(bt, x0pb)
        out1_ref[...] = jax.lax.slice(x1p, (0, 0), (n_edges, ch))
        y2 = _xw_mm(x1p, w1_ref[...])
        out0T = _mm(_tb(y2), bt)
        out0_ref[...] = jnp.swapaxes(out0T, 0, 1)


def kernel(x_0, incidence_1, W0, W1):
    n_nodes, ch = x_0.shape
    n_edges = incidence_1.shape[1]
    bt = jnp.swapaxes(incidence_1, 0, 1)
    x0b = x_0.astype(jnp.bfloat16)
    return pl.pallas_call(
        _body,
        grid=(_NF + 2,),
        in_specs=[
            pl.BlockSpec((n_nodes, ch), lambda i: (0, 0)),
            pl.BlockSpec((_CR, n_nodes), lambda i: (jnp.minimum(i, _NF), 0)),
            pl.BlockSpec((ch, ch), lambda i: (0, 0)),
            pl.BlockSpec((ch, ch), lambda i: (0, 0)),
        ],
        out_specs=(
            pl.BlockSpec((n_nodes, ch), lambda i: (0, 0)),
            pl.BlockSpec((n_edges, ch), lambda i: (0, 0)),
        ),
        out_shape=(
            jax.ShapeDtypeStruct((n_nodes, ch), jnp.float32),
            jax.ShapeDtypeStruct((n_edges, ch), jnp.float32),
        ),
        scratch_shapes=[
            pltpu.VMEM((_EP, n_nodes), jnp.bfloat16),
        ],
    )(x0b, bt, W0, W1)


# speedup vs baseline: 1.1328x; 1.0751x over previous
import jax
import jax.numpy as jnp
from jax.experimental import pallas as pl
from jax.experimental.pallas import tpu as pltpu

_CR = 80
_NF = 12
_EP = 1008


def _mm(a, b):
    dn = (((1,), (0,)), ((), ()))
    return jax.lax.dot_general(a, b, dn, preferred_element_type=jnp.float32)


def _xw_mm(x, w):
    xh = x.astype(jnp.bfloat16)
    xl = (x - xh.astype(jnp.float32)).astype(jnp.bfloat16)
    wh = w.astype(jnp.bfloat16)
    wl = (w - wh.astype(jnp.float32)).astype(jnp.bfloat16)
    return _mm(xh, wh) + _mm(xh, wl) + _mm(xl, wh)


def _tb(v):
    return jnp.swapaxes(v.astype(jnp.bfloat16), 0, 1)


def _body(x0_ref, bt_ref, w0_ref, w1_ref, out0_ref, out1_ref, btc_ref, x0b_ref):
    i = pl.program_id(0)
    n_nodes = bt_ref.shape[1]
    ch = x0_ref.shape[1]
    n_edges = out1_ref.shape[0]

    @pl.when(i == 0)
    def _castx0():
        x0b_ref[...] = x0_ref[...].astype(jnp.bfloat16)

    @pl.when(i < _NF)
    def _build():
        btc_ref[pl.ds(i * _CR, _CR), :] = bt_ref[...].astype(jnp.bfloat16)

    @pl.when(i == _NF)
    def _tail():
        tail = n_edges - _NF * _CR
        blk = bt_ref[pl.ds(0, tail), :].astype(jnp.bfloat16)
        btc_ref[pl.ds(_NF * _CR, _EP - _NF * _CR), :] = jnp.concatenate(
            [blk, jnp.zeros((_EP - n_edges, n_nodes), jnp.bfloat16)], axis=0)

    @pl.when(i == _NF + 1)
    def _compute():
        bt = btc_ref[...]
        x1 = _mm(bt, x0b_ref[...])
        y = _xw_mm(x1, w0_ref[...])
        x0pT = _mm(_tb(y), bt)
        x0pb = jnp.swapaxes(x0pT.astype(jnp.bfloat16), 0, 1)
        x1p = _mm(bt, x0pb)
        out1_ref[...] = jax.lax.slice(x1p, (0, 0), (n_edges, ch))
        y2 = _xw_mm(x1p, w1_ref[...])
        out0T = _mm(_tb(y2), bt)
        out0_ref[...] = jnp.swapaxes(out0T, 0, 1)


def kernel(x_0, incidence_1, W0, W1):
    n_nodes, ch = x_0.shape
    n_edges = incidence_1.shape[1]
    bt = jnp.swapaxes(incidence_1, 0, 1)
    return pl.pallas_call(
        _body,
        grid=(_NF + 2,),
        in_specs=[
            pl.BlockSpec((n_nodes, ch), lambda i: (0, 0)),
            pl.BlockSpec((_CR, n_nodes), lambda i: (jnp.minimum(i, _NF), 0)),
            pl.BlockSpec((ch, ch), lambda i: (0, 0)),
            pl.BlockSpec((ch, ch), lambda i: (0, 0)),
        ],
        out_specs=(
            pl.BlockSpec((n_nodes, ch), lambda i: (0, 0)),
            pl.BlockSpec((n_edges, ch), lambda i: (0, 0)),
        ),
        out_shape=(
            jax.ShapeDtypeStruct((n_nodes, ch), jnp.float32),
            jax.ShapeDtypeStruct((n_edges, ch), jnp.float32),
        ),
        scratch_shapes=[
            pltpu.VMEM((_EP, n_nodes), jnp.bfloat16),
            pltpu.VMEM((n_nodes, ch), jnp.bfloat16),
        ],
        compiler_params=pltpu.CompilerParams(vmem_limit_bytes=66584576),
    )(x_0, bt, W0, W1)
